# Initial kernel scaffold; baseline (speedup 1.0000x reference)
import jax
import jax.numpy as jnp
from jax.experimental import pallas as pl


def kernel(x, edge_index, W1l, b1l, W1r, bn_gamma, bn_beta, W2l, b2l, W2r):
    raise NotImplementedError("write your pallas kernel here")


# speedup vs baseline: 10.4779x; 10.4779x over previous
import functools

import jax
import jax.numpy as jnp
from jax import lax
from jax.experimental import pallas as pl
from jax.experimental.pallas import tpu as pltpu
from jax.experimental.pallas import tpu_sc as plsc

N = 10000
E = 320000
D = 128
NC = 2
NS = 16
NW = NC * NS
EPT = E // NW
CHUNK = 125
NCHUNK = EPT // CHUNK
RPT = N // NS
RCH = 125
NRCH = RPT // RCH
CW = 16


def _sc_agg_body(with_cnt, *refs):
    if with_cnt:
        (x_hbm, src_hbm, dst_hbm, agg_out, cnt_out,
         sidx, didx, buf_a, buf_b, ones_v, agg_sh, cnt_sh, sem_i, sem_g) = refs
    else:
        (x_hbm, src_hbm, dst_hbm, agg_out,
         sidx, didx, buf_a, buf_b, agg_sh, sem_i, sem_g) = refs

    c = lax.axis_index("c")
    s = lax.axis_index("s")
    wid = c * NS + s

    z16 = jnp.zeros((16,), jnp.float32)

    def zero_row(i, _):
        r = i // (D // 16)
        col = (i % (D // 16)) * 16
        buf_a[r, pl.ds(col, 16)] = z16
        return 0

    lax.fori_loop(0, RCH * (D // 16), zero_row, 0)
    for k in range(NRCH):
        pltpu.sync_copy(buf_a, agg_sh.at[pl.ds(s * RPT + k * RCH, RCH)])

    if with_cnt:
        def zero_cnt_row(i, _):
            ones_v[i] = z16
            return 0

        lax.fori_loop(0, RCH, zero_cnt_row, 0)
        for k in range(NRCH):
            pltpu.sync_copy(ones_v, cnt_sh.at[pl.ds(s * RPT + k * RCH, RCH)])

        o16 = jnp.ones((16,), jnp.float32)

        def fill_ones(i, _):
            ones_v[i] = o16
            return 0

        lax.fori_loop(0, RCH, fill_ones, 0)

    plsc.subcore_barrier()

    def idx_copy(j, slot):
        return (pltpu.make_async_copy(src_hbm.at[wid, j], sidx.at[slot], sem_i),
                pltpu.make_async_copy(dst_hbm.at[wid, j], didx.at[slot], sem_i))

    def gather_copy(slot, buf):
        return pltpu.make_async_copy(x_hbm.at[sidx.at[slot]], buf, sem_g)

    for d in idx_copy(0, 0):
        d.start()
    for d in idx_copy(0, 0):
        d.wait()
    gather_copy(0, buf_a).start()
    for d in idx_copy(1, 1):
        d.start()

    def chunk_step(j, b, cur_buf, nxt_buf):
        gather_copy(b, cur_buf).wait()

        @pl.when(j + 1 < NCHUNK)
        def _():
            for d in idx_copy(j + 1, 1 - b):
                d.wait()
            gather_copy(1 - b, nxt_buf).start()

        pltpu.sync_copy(cur_buf, agg_sh.at[didx.at[b]], add=True)
        if with_cnt:
            pltpu.sync_copy(ones_v, cnt_sh.at[didx.at[b]], add=True)

        @pl.when(j + 2 < NCHUNK)
        def _():
            for d in idx_copy(j + 2, b):
                d.start()

    def edge_pair(g, _):
        chunk_step(2 * g, 0, buf_a, buf_b)
        chunk_step(2 * g + 1, 1, buf_b, buf_a)
        return 0

    lax.fori_loop(0, NCHUNK // 2, edge_pair, 0)

    plsc.subcore_barrier()

    for k in range(NRCH):
        sl = pl.ds(s * RPT + k * RCH, RCH)
        pltpu.sync_copy(agg_sh.at[sl], agg_out.at[c, sl])
        if with_cnt:
            pltpu.sync_copy(cnt_sh.at[sl], cnt_out.at[c, sl])


def _make_sc_agg(with_cnt):
    mesh = plsc.VectorSubcoreMesh(core_axis_name="c", subcore_axis_name="s")
    out_type = [jax.ShapeDtypeStruct((NC, N, D), jnp.float32)]
    scratch = [
        pltpu.VMEM((2, CHUNK), jnp.int32),
        pltpu.VMEM((2, CHUNK), jnp.int32),
        pltpu.VMEM((RCH, D), jnp.float32),
        pltpu.VMEM((RCH, D), jnp.float32),
    ]
    if with_cnt:
        out_type.append(jax.ShapeDtypeStruct((NC, N, CW), jnp.float32))
        scratch.append(pltpu.VMEM((RCH, CW), jnp.float32))
    scratch.append(pltpu.VMEM_SHARED((N, D), jnp.float32))
    if with_cnt:
        scratch.append(pltpu.VMEM_SHARED((N, CW), jnp.float32))
    scratch.append(pltpu.SemaphoreType.DMA)
    scratch.append(pltpu.SemaphoreType.DMA)

    return pl.kernel(
        functools.partial(_sc_agg_body, with_cnt),
        out_type=tuple(out_type),
        mesh=mesh,
        scratch_types=tuple(scratch),
        compiler_params=pltpu.CompilerParams(use_tc_tiling_on_sc=False),
    )


def _tc_layer1(aggp, cntp, x, w1lt, b1l, w1rt, gamma, beta, h_out, inv_out):
    agg = aggp[0] + aggp[1]
    cnt = cntp[0, :, 0:1] + cntp[1, :, 0:1]
    inv = 1.0 / jnp.maximum(cnt, 1.0)
    out = (jnp.dot(agg * inv, w1lt[...], preferred_element_type=jnp.float32)
           + b1l[...]
           + jnp.dot(x[...], w1rt[...], preferred_element_type=jnp.float32))
    norm = jnp.sqrt(jnp.sum(out * out, axis=1, keepdims=True))
    out = out / jnp.maximum(norm, 1e-12)
    mean = jnp.mean(out, axis=0, keepdims=True)
    var = jnp.mean((out - mean) ** 2, axis=0, keepdims=True)
    out = (out - mean) / jnp.sqrt(var + 1e-5) * gamma[...] + beta[...]
    h_out[...] = jnp.maximum(out, 0.0)
    inv_out[...] = inv


def _tc_layer2(aggp, inv, h, w2lt, b2l, w2rt, out_ref):
    agg = (aggp[0] + aggp[1]) * inv[...]
    out = (jnp.dot(agg, w2lt[...], preferred_element_type=jnp.float32)
           + b2l[...]
           + jnp.dot(h[...], w2rt[...], preferred_element_type=jnp.float32))
    norm = jnp.sqrt(jnp.sum(out * out, axis=1, keepdims=True))
    out_ref[...] = out / jnp.maximum(norm, 1e-12)


def kernel(x, edge_index, W1l, b1l, W1r, bn_gamma, bn_beta, W2l, b2l, W2r):
    src3 = edge_index[0].reshape(NW, NCHUNK, CHUNK)
    dst3 = edge_index[1].reshape(NW, NCHUNK, CHUNK)

    agg1, cnt1 = _make_sc_agg(True)(x, src3, dst3)

    h, inv = pl.pallas_call(
        _tc_layer1,
        out_shape=(
            jax.ShapeDtypeStruct((N, D), jnp.float32),
            jax.ShapeDtypeStruct((N, 1), jnp.float32),
        ),
    )(agg1, cnt1, x, W1l.T, b1l.reshape(1, D), W1r.T,
      bn_gamma.reshape(1, D), bn_beta.reshape(1, D))

    (agg2,) = _make_sc_agg(False)(h, src3, dst3)

    out = pl.pallas_call(
        _tc_layer2,
        out_shape=jax.ShapeDtypeStruct((N, D), jnp.float32),
    )(agg2, inv, h, W2l.T, b2l.reshape(1, D), W2r.T)
    return out
